# Initial kernel scaffold; baseline (speedup 1.0000x reference)
import jax
import jax.numpy as jnp
from jax.experimental import pallas as pl


def kernel(lhs, rhs):
    raise NotImplementedError("write your pallas kernel here")


# speedup vs baseline: 6.8779x; 6.8779x over previous
import jax
import jax.numpy as jnp
from jax.experimental import pallas as pl


def _l2_band_kernel(lhs_ref, rhs_ref, out_ref, *, n_col_blocks, col_block):
    j = pl.program_id(0)
    B, N, _ = out_ref.shape
    fill = jnp.full(out_ref.shape, -1000000000.0, dtype=out_ref.dtype)

    @pl.when(j == n_col_blocks - 1)
    def _last():
        diff = lhs_ref[...] - rhs_ref[...]
        sim = jnp.sqrt(jnp.sum(diff * diff, axis=-1))
        row = jax.lax.broadcasted_iota(jnp.int32, (B, N, col_block), 1)
        col = jax.lax.broadcasted_iota(jnp.int32, (B, N, col_block), 2)
        mask = col == row + (col_block - N)
        out_ref[...] = jnp.where(mask, sim[:, :, None], fill)

    @pl.when(j != n_col_blocks - 1)
    def _rest():
        out_ref[...] = fill


def kernel(lhs, rhs):
    B, N, dim = lhs.shape
    N_ = rhs.shape[1]
    col_block = 512
    n_col_blocks = N_ // col_block
    tail_block_idx = N_ // N - 1

    import functools
    body = functools.partial(
        _l2_band_kernel, n_col_blocks=n_col_blocks, col_block=col_block
    )
    return pl.pallas_call(
        body,
        grid=(n_col_blocks,),
        in_specs=[
            pl.BlockSpec((B, N, dim), lambda j: (0, 0, 0)),
            pl.BlockSpec((B, N, dim), lambda j: (0, tail_block_idx, 0)),
        ],
        out_specs=pl.BlockSpec((B, N, col_block), lambda j: (0, 0, j)),
        out_shape=jax.ShapeDtypeStruct((B, N, N_), lhs.dtype),
    )(lhs, rhs)


# speedup vs baseline: 7.5837x; 1.1026x over previous
import functools

import jax
import jax.numpy as jnp
from jax.experimental import pallas as pl


def _l2_band_kernel(lhs_ref, rhs_ref, out_ref, *, tail):
    bb, N, N_ = out_ref.shape
    out_ref[...] = jnp.full(out_ref.shape, -1000000000.0, dtype=out_ref.dtype)
    diff = lhs_ref[...] - rhs_ref[...]
    sim = jnp.sqrt(jnp.sum(diff * diff, axis=-1))
    row = jax.lax.broadcasted_iota(jnp.int32, (bb, N, tail), 1)
    col = jax.lax.broadcasted_iota(jnp.int32, (bb, N, tail), 2)
    mask = col == row + (tail - N)
    out_ref[:, :, N_ - tail:] = jnp.where(
        mask, sim[:, :, None], jnp.float32(-1000000000.0)
    )


def kernel(lhs, rhs):
    B, N, dim = lhs.shape
    N_ = rhs.shape[1]
    bb = 4
    tail = 128
    tail_block_idx = N_ // N - 1

    body = functools.partial(_l2_band_kernel, tail=tail)
    return pl.pallas_call(
        body,
        grid=(B // bb,),
        in_specs=[
            pl.BlockSpec((bb, N, dim), lambda j: (j, 0, 0)),
            pl.BlockSpec((bb, N, dim), lambda j: (j, tail_block_idx, 0)),
        ],
        out_specs=pl.BlockSpec((bb, N, N_), lambda j: (j, 0, 0)),
        out_shape=jax.ShapeDtypeStruct((B, N, N_), lhs.dtype),
    )(lhs, rhs)
